# baseline (device time: 16680 ns/iter reference)
import jax
import jax.numpy as jnp
from jax import lax
from jax.experimental import pallas as pl
from jax.experimental.pallas import tpu as pltpu

N_DEV = 4
N_TOK = 512
D_IN = 256
D_OUT = 512
N_EXP = 8
E_PER_SHARD = 2
CAPACITY = 51
CHUNK = N_TOK // N_DEV


def kernel(x, router_W, route_idx, expert_W):
    del router_W

    my_pos = lax.axis_index("i")

    onehot = route_idx == jnp.arange(N_EXP, dtype=jnp.int32)[None, :]
    cum = jnp.cumsum(onehot.astype(jnp.int32), axis=0)
    keep = (onehot & (cum <= CAPACITY)).astype(jnp.float32)
    keep_local = lax.dynamic_slice(
        keep, (0, my_pos * E_PER_SHARD), (N_TOK, E_PER_SHARD)
    )

    def body(x_ref, keep_ref, w_ref, out_ref, partial_ref, recv_ref,
             send_sems, recv_sems):
        pos = lax.axis_index("i")

        barrier_sem = pltpu.get_barrier_semaphore()
        for j in range(1, N_DEV):
            pl.semaphore_signal(
                barrier_sem, inc=1,
                device_id=((pos + j) % N_DEV,),
                device_id_type=pl.DeviceIdType.MESH,
            )
        pl.semaphore_wait(barrier_sem, N_DEV - 1)

        xv = x_ref[...]
        acc = jnp.dot(
            xv * keep_ref[:, 0:1], w_ref[0],
            preferred_element_type=jnp.float32,
        )
        acc += jnp.dot(
            xv * keep_ref[:, 1:2], w_ref[1],
            preferred_element_type=jnp.float32,
        )
        partial_ref[...] = acc

        rdmas = []
        for j in range(N_DEV - 1):
            r = (pos + 1 + j) % N_DEV
            slot = N_DEV - 2 - j
            rdma = pltpu.make_async_remote_copy(
                src_ref=partial_ref.at[pl.ds(r * CHUNK, CHUNK), :],
                dst_ref=recv_ref.at[slot],
                send_sem=send_sems.at[j],
                recv_sem=recv_sems.at[slot],
                device_id=(r,),
                device_id_type=pl.DeviceIdType.MESH,
            )
            rdma.start()
            rdmas.append(rdma)
        for rdma in rdmas:
            rdma.wait()

        out_ref[...] = (
            partial_ref[pl.ds(pos * CHUNK, CHUNK), :]
            + recv_ref[0] + recv_ref[1] + recv_ref[2]
        )

    return pl.pallas_call(
        body,
        out_shape=jax.ShapeDtypeStruct((CHUNK, D_OUT), jnp.float32),
        in_specs=[
            pl.BlockSpec(memory_space=pltpu.VMEM),
            pl.BlockSpec(memory_space=pltpu.VMEM),
            pl.BlockSpec(memory_space=pltpu.VMEM),
        ],
        out_specs=pl.BlockSpec(memory_space=pltpu.VMEM),
        scratch_shapes=[
            pltpu.VMEM((N_TOK, D_OUT), jnp.float32),
            pltpu.VMEM((N_DEV - 1, CHUNK, D_OUT), jnp.float32),
            pltpu.SemaphoreType.DMA((N_DEV - 1,)),
            pltpu.SemaphoreType.DMA((N_DEV - 1,)),
        ],
        compiler_params=pltpu.CompilerParams(collective_id=0),
    )(x, keep_local, expert_W)


# device time: 15294 ns/iter; 1.0906x vs baseline; 1.0906x over previous
import jax
import jax.numpy as jnp
from jax import lax
from jax.experimental import pallas as pl
from jax.experimental.pallas import tpu as pltpu

N_DEV = 4
N_TOK = 512
D_IN = 256
D_OUT = 512
E_PER_SHARD = 2
CAPACITY = 51
CHUNK = N_TOK // N_DEV

_SEND_OFFSETS = (2, 1, 3)


def kernel(x, router_W, route_idx, expert_W):
    del router_W

    def body(x_ref, idx_ref, w_ref, out_ref, xm_ref, send_ref, recv_ref,
             send_sems, recv_sems):
        pos = lax.axis_index("i")

        barrier_sem = pltpu.get_barrier_semaphore()
        for o in range(1, N_DEV):
            pl.semaphore_signal(
                barrier_sem, inc=1,
                device_id=((pos + o) % N_DEV,),
                device_id_type=pl.DeviceIdType.MESH,
            )
        pl.semaphore_wait(barrier_sem, N_DEV - 1)

        ex = pos * E_PER_SHARD + lax.broadcasted_iota(
            jnp.int32, (1, E_PER_SHARD), 1
        )
        m = idx_ref[...] == ex
        row = lax.broadcasted_iota(jnp.int32, (N_TOK, N_TOK), 0)
        col = lax.broadcasted_iota(jnp.int32, (N_TOK, N_TOK), 1)
        tri = (col <= row).astype(jnp.float32)
        cum = jnp.dot(tri, m.astype(jnp.float32),
                      preferred_element_type=jnp.float32)
        keep = jnp.where(m & (cum <= float(CAPACITY)), 1.0, 0.0)

        xv = x_ref[...]
        xm_ref[0] = xv * keep[:, 0:1]
        xm_ref[1] = xv * keep[:, 1:2]

        rdmas = []
        for j, o in enumerate(_SEND_OFFSETS):
            r = (pos + o) % N_DEV
            rows = pl.ds(r * CHUNK, CHUNK)
            send_ref[j] = jnp.dot(
                xm_ref[0, rows, :], w_ref[0],
                preferred_element_type=jnp.float32,
            ) + jnp.dot(
                xm_ref[1, rows, :], w_ref[1],
                preferred_element_type=jnp.float32,
            )
            rdma = pltpu.make_async_remote_copy(
                src_ref=send_ref.at[j],
                dst_ref=recv_ref.at[3 - o],
                send_sem=send_sems.at[j],
                recv_sem=recv_sems.at[3 - o],
                device_id=(r,),
                device_id_type=pl.DeviceIdType.MESH,
            )
            rdma.start()
            rdmas.append(rdma)

        rows = pl.ds(pos * CHUNK, CHUNK)
        own = jnp.dot(
            xm_ref[0, rows, :], w_ref[0], preferred_element_type=jnp.float32
        ) + jnp.dot(
            xm_ref[1, rows, :], w_ref[1], preferred_element_type=jnp.float32
        )

        for rdma in rdmas:
            rdma.wait_recv()
        out_ref[...] = own + recv_ref[0] + recv_ref[1] + recv_ref[2]
        for rdma in rdmas:
            rdma.wait_send()

    return pl.pallas_call(
        body,
        out_shape=jax.ShapeDtypeStruct((CHUNK, D_OUT), jnp.float32),
        in_specs=[
            pl.BlockSpec(memory_space=pltpu.VMEM),
            pl.BlockSpec(memory_space=pltpu.VMEM),
            pl.BlockSpec(memory_space=pltpu.VMEM),
        ],
        out_specs=pl.BlockSpec(memory_space=pltpu.VMEM),
        scratch_shapes=[
            pltpu.VMEM((E_PER_SHARD, N_TOK, D_IN), jnp.float32),
            pltpu.VMEM((N_DEV - 1, CHUNK, D_OUT), jnp.float32),
            pltpu.VMEM((N_DEV - 1, CHUNK, D_OUT), jnp.float32),
            pltpu.SemaphoreType.DMA((N_DEV - 1,)),
            pltpu.SemaphoreType.DMA((N_DEV - 1,)),
        ],
        compiler_params=pltpu.CompilerParams(collective_id=0),
    )(x, route_idx, expert_W)


# device time: 12549 ns/iter; 1.3292x vs baseline; 1.2187x over previous
import jax
import jax.numpy as jnp
from jax import lax
from jax.experimental import pallas as pl
from jax.experimental.pallas import tpu as pltpu

N_DEV = 4
N_TOK = 512
D_IN = 256
D_OUT = 512
E_PER_SHARD = 2
CAPACITY = 51
CHUNK = N_TOK // N_DEV

_SEND_OFFSETS = (2, 1, 3)


def kernel(x, router_W, route_idx, expert_W):
    del router_W

    def body(x_ref, idx_ref, w_ref, out_ref, keep_ref, send_ref, recv_ref,
             send_sems, recv_sems):
        pos = lax.axis_index("i")

        barrier_sem = pltpu.get_barrier_semaphore()
        for o in range(1, N_DEV):
            pl.semaphore_signal(
                barrier_sem, inc=1,
                device_id=((pos + o) % N_DEV,),
                device_id_type=pl.DeviceIdType.MESH,
            )
        pl.semaphore_wait(barrier_sem, N_DEV - 1)

        ex = pos * E_PER_SHARD + lax.broadcasted_iota(
            jnp.int32, (1, E_PER_SHARD), 1
        )
        m = idx_ref[...] == ex
        row = lax.broadcasted_iota(jnp.int32, (N_TOK, N_TOK), 0)
        col = lax.broadcasted_iota(jnp.int32, (N_TOK, N_TOK), 1)
        tri = (col <= row).astype(jnp.float32)
        cum = jnp.dot(tri, m.astype(jnp.float32),
                      preferred_element_type=jnp.float32)
        keep_ref[...] = jnp.where(m & (cum <= float(CAPACITY)), 1.0, 0.0)

        def chunk_out(rows):
            xr = x_ref[rows, :]
            return jnp.dot(
                xr * keep_ref[rows, 0:1], w_ref[0],
                preferred_element_type=jnp.float32,
            ) + jnp.dot(
                xr * keep_ref[rows, 1:2], w_ref[1],
                preferred_element_type=jnp.float32,
            )

        rdmas = []
        for j, o in enumerate(_SEND_OFFSETS):
            r = (pos + o) % N_DEV
            send_ref[j] = chunk_out(pl.ds(r * CHUNK, CHUNK)).astype(
                jnp.bfloat16
            )
            rdma = pltpu.make_async_remote_copy(
                src_ref=send_ref.at[j],
                dst_ref=recv_ref.at[3 - o],
                send_sem=send_sems.at[j],
                recv_sem=recv_sems.at[3 - o],
                device_id=(r,),
                device_id_type=pl.DeviceIdType.MESH,
            )
            rdma.start()
            rdmas.append(rdma)

        own = chunk_out(pl.ds(pos * CHUNK, CHUNK))

        for rdma in rdmas:
            rdma.wait_recv()
        out_ref[...] = (
            own
            + recv_ref[0].astype(jnp.float32)
            + recv_ref[1].astype(jnp.float32)
            + recv_ref[2].astype(jnp.float32)
        )
        for rdma in rdmas:
            rdma.wait_send()

    return pl.pallas_call(
        body,
        out_shape=jax.ShapeDtypeStruct((CHUNK, D_OUT), jnp.float32),
        in_specs=[
            pl.BlockSpec(memory_space=pltpu.VMEM),
            pl.BlockSpec(memory_space=pltpu.VMEM),
            pl.BlockSpec(memory_space=pltpu.VMEM),
        ],
        out_specs=pl.BlockSpec(memory_space=pltpu.VMEM),
        scratch_shapes=[
            pltpu.VMEM((N_TOK, E_PER_SHARD), jnp.float32),
            pltpu.VMEM((N_DEV - 1, CHUNK, D_OUT), jnp.bfloat16),
            pltpu.VMEM((N_DEV - 1, CHUNK, D_OUT), jnp.bfloat16),
            pltpu.SemaphoreType.DMA((N_DEV - 1,)),
            pltpu.SemaphoreType.DMA((N_DEV - 1,)),
        ],
        compiler_params=pltpu.CompilerParams(collective_id=0),
    )(x, route_idx, expert_W)
